# baseline (speedup 1.0000x reference)
import jax
import jax.numpy as jnp
from jax.experimental import pallas as pl

B, S, K, D = 1024, 50, 20, 128
BB = 128
R2 = BB * K


def _entity_kernel(x_ref, bias_ref, keys_ref, P_ref, U_ref, V_ref, W_ref,
                   out_ref):
    keys2 = keys_ref[...]
    P = P_ref[...]
    U = U_ref[...]
    V = V_ref[...]
    W = W_ref[...]

    keysV = jnp.dot(keys2, V, preferred_element_type=jnp.float32)

    def step(t, h):
        x_t = x_ref[t]
        bias_t = bias_ref[t].reshape(R2)
        xW = jnp.dot(x_t, W, preferred_element_type=jnp.float32)
        x_aug = jnp.concatenate([x_t, xW], axis=1)
        xb_aug = jnp.dot(P, x_aug, preferred_element_type=jnp.float32)
        xb = xb_aug[:, :D]
        xWb = xb_aug[:, D:]
        z = jnp.sum(xb * (h + keys2), axis=1)
        g = 0.5 * jnp.tanh(0.5 * z + bias_t) + 0.5
        hU = jnp.dot(h, U, preferred_element_type=jnp.float32)
        h_tilda = jax.nn.relu(hU + keysV + xWb)
        upd = h + g[:, None] * h_tilda
        inv = jax.lax.rsqrt(jnp.maximum(
            jnp.sum(upd * upd, axis=1, keepdims=True), 1e-12))
        return upd * inv

    h0 = jnp.zeros((R2, D), dtype=jnp.float32)
    out_ref[...] = jax.lax.fori_loop(0, S, step, h0)


@jax.jit
def kernel(encoded_sents, mask, keys, U, V, W):
    x_t_first = jnp.swapaxes(encoded_sents, 0, 1)
    bias = ((jnp.swapaxes(mask, 0, 1).astype(jnp.float32) - 1.0) * 5000.0)
    bias = jnp.repeat(bias, K, axis=1)[:, None, :]
    keys2 = keys.reshape(B * K, D)
    P = jnp.repeat(jnp.eye(BB, dtype=jnp.float32), K, axis=0)
    grid = (B // BB,)
    out2 = pl.pallas_call(
        _entity_kernel,
        grid=grid,
        in_specs=[
            pl.BlockSpec((S, BB, D), lambda i: (0, i, 0)),
            pl.BlockSpec((S, 1, R2), lambda i: (0, 0, i)),
            pl.BlockSpec((R2, D), lambda i: (i, 0)),
            pl.BlockSpec((R2, BB), lambda i: (0, 0)),
            pl.BlockSpec((D, D), lambda i: (0, 0)),
            pl.BlockSpec((D, D), lambda i: (0, 0)),
            pl.BlockSpec((D, D), lambda i: (0, 0)),
        ],
        out_specs=pl.BlockSpec((R2, D), lambda i: (i, 0)),
        out_shape=jax.ShapeDtypeStruct((B * K, D), jnp.float32),
    )(x_t_first, bias, keys2, P, U, V, W)
    return out2.reshape(B, K, D)


# speedup vs baseline: 2.2001x; 2.2001x over previous
import jax
import jax.numpy as jnp
from jax.experimental import pallas as pl

B, S, K, D = 1024, 50, 20, 128
K2 = 24
BB = 128


def _entity_kernel(x_ref, m_ref, keys_ref, U_ref, V_ref, W_ref, out_ref):
    keys = keys_ref[...]
    U = U_ref[...]
    V = V_ref[...]
    W = W_ref[...]

    keysV = jnp.dot(keys.reshape(BB * K2, D), V,
                    preferred_element_type=jnp.float32).reshape(BB, K2, D)

    def step(t, h):
        x_t = x_ref[t]
        m_t = m_ref[t].reshape(BB, 1)
        z = jnp.sum(x_t[:, None, :] * (h + keys), axis=2)
        g = m_t * (0.5 * jnp.tanh(0.5 * z) + 0.5)
        hU = jnp.dot(h.reshape(BB * K2, D), U,
                     preferred_element_type=jnp.float32).reshape(BB, K2, D)
        xW = jnp.dot(x_t, W, preferred_element_type=jnp.float32)
        h_tilda = jax.nn.relu(hU + keysV + xW[:, None, :])
        upd = h + g[..., None] * h_tilda
        inv = jax.lax.rsqrt(jnp.maximum(
            jnp.sum(upd * upd, axis=2, keepdims=True), 1e-12))
        return upd * inv

    h0 = jnp.zeros((BB, K2, D), dtype=jnp.float32)
    h_final = jax.lax.fori_loop(0, S, step, h0)
    out_ref[...] = h_final[:, :K, :]


@jax.jit
def kernel(encoded_sents, mask, keys, U, V, W):
    x_t_first = jnp.swapaxes(encoded_sents, 0, 1)
    mask_f = jnp.swapaxes(mask, 0, 1).astype(jnp.float32)[:, None, :]
    keys_p = jnp.pad(keys, ((0, 0), (0, K2 - K), (0, 0)))
    grid = (B // BB,)
    return pl.pallas_call(
        _entity_kernel,
        grid=grid,
        in_specs=[
            pl.BlockSpec((S, BB, D), lambda i: (0, i, 0)),
            pl.BlockSpec((S, 1, BB), lambda i: (0, 0, i)),
            pl.BlockSpec((BB, K2, D), lambda i: (i, 0, 0)),
            pl.BlockSpec((D, D), lambda i: (0, 0)),
            pl.BlockSpec((D, D), lambda i: (0, 0)),
            pl.BlockSpec((D, D), lambda i: (0, 0)),
        ],
        out_specs=pl.BlockSpec((BB, K, D), lambda i: (i, 0, 0)),
        out_shape=jax.ShapeDtypeStruct((B, K, D), jnp.float32),
    )(x_t_first, mask_f, keys_p, U, V, W)
